# baseline (speedup 1.0000x reference)
import functools

import jax
import jax.numpy as jnp
from jax import lax
from jax.experimental import pallas as pl
from jax.experimental.pallas import tpu as pltpu, tpu_sc as plsc

HIDDEN = 2048
NUM_EXPERTS = 16
TOP_K = 2
N_TOKENS = 4 * 4096

CHUNK = 256
NBUF = 8
NSPLIT = 2
NOUT = 4

_NC, _NS = 2, 16
_NW = _NC * _NS
_ROWS_PER_W = N_TOKENS // _NW
_TILES = _ROWS_PER_W // 16


def _matmul_body(x_hbm, wt_ref, logits_hbm, buf, sem, st_l, osem):
    n_chunks = x_hbm.shape[0] // CHUNK
    csz = HIDDEN // NSPLIT

    def start_copy(i, slot):
        for j in range(NSPLIT):
            pltpu.make_async_copy(
                x_hbm.at[pl.ds(i * CHUNK, CHUNK), pl.ds(j * csz, csz)],
                buf.at[slot, slice(None), pl.ds(j * csz, csz)],
                sem.at[slot, j],
            ).start()

    def wait_copy(slot):
        for j in range(NSPLIT):
            pltpu.make_async_copy(
                x_hbm.at[pl.ds(0, CHUNK), pl.ds(0, csz)],
                buf.at[slot, slice(None), pl.ds(j * csz, csz)],
                sem.at[slot, j],
            ).wait()

    def out_copy(i, oslot):
        return pltpu.make_async_copy(
            st_l.at[oslot], logits_hbm.at[pl.ds(i * CHUNK, CHUNK), :],
            osem.at[oslot])

    for s in range(NBUF):
        start_copy(s, s)

    wt = wt_ref[...]

    def chunk_body(i, _):
        slot = lax.rem(i, NBUF)
        oslot = lax.rem(i, NOUT)
        wait_copy(slot)
        logits = jax.lax.dot_general(
            buf[slot], wt, (((1,), (0,)), ((), ())),
            preferred_element_type=jnp.float32)

        @pl.when(i + NBUF < n_chunks)
        def _():
            start_copy(i + NBUF, slot)

        @pl.when(i >= NOUT)
        def _():
            out_copy(i - NOUT, oslot).wait()

        st_l[oslot] = logits
        out_copy(i, oslot).start()
        return 0

    lax.fori_loop(0, n_chunks, chunk_body, 0)
    for k in range(NOUT):
        i = n_chunks - NOUT + k
        out_copy(i, lax.rem(jnp.int32(i), NOUT)).wait()


def _routing_body(logits_hbm, probs_hbm, w_hbm, i_hbm, lv, pv, wv, iv):
    wid = lax.axis_index("c") * _NS + lax.axis_index("s")
    base = wid * _ROWS_PER_W
    nwords = _ROWS_PER_W * NUM_EXPERTS
    pltpu.sync_copy(logits_hbm.at[pl.ds(base * NUM_EXPERTS, nwords)], lv)

    def tile_body(t, _):
        iota = lax.iota(jnp.int32, 16)
        fidx0 = t * (16 * NUM_EXPERTS) + iota * NUM_EXPERTS
        regs = [plsc.load_gather(lv, [fidx0 + e]) for e in range(NUM_EXPERTS)]
        m = regs[0]
        for e in range(1, NUM_EXPERTS):
            m = jnp.maximum(m, regs[e])
        es = [jnp.exp(r - m) for r in regs]
        s = es[0]
        for e in range(1, NUM_EXPERTS):
            s = s + es[e]
        inv = 1.0 / s
        ps = [ee * inv for ee in es]
        for e in range(NUM_EXPERTS):
            plsc.store_scatter(pv, [fidx0 + e], ps[e])
        m1 = ps[0]
        i1 = jnp.zeros((16,), jnp.int32)
        m2 = jnp.full((16,), -1.0, jnp.float32)
        i2 = jnp.zeros((16,), jnp.int32)
        for e in range(1, NUM_EXPERTS):
            ev = jnp.full((16,), e, jnp.int32)
            gt1 = ps[e] > m1
            gt2 = ps[e] > m2
            m2 = jnp.where(gt1, m1, jnp.where(gt2, ps[e], m2))
            i2 = jnp.where(gt1, i1, jnp.where(gt2, ev, i2))
            m1 = jnp.where(gt1, ps[e], m1)
            i1 = jnp.where(gt1, ev, i1)
        denom = m1 + m2
        li2 = t * (16 * TOP_K) + iota * TOP_K
        li2p = li2 + 1
        plsc.store_scatter(wv, [li2], m1 / denom)
        plsc.store_scatter(wv, [li2p], m2 / denom)
        plsc.store_scatter(iv, [li2], i1)
        plsc.store_scatter(iv, [li2p], i2)
        return 0

    lax.fori_loop(0, _TILES, tile_body, 0)
    pltpu.sync_copy(pv, probs_hbm.at[pl.ds(base * NUM_EXPERTS, nwords)])
    pltpu.sync_copy(wv, w_hbm.at[pl.ds(TOP_K * base, TOP_K * _ROWS_PER_W)])
    pltpu.sync_copy(iv, i_hbm.at[pl.ds(TOP_K * base, TOP_K * _ROWS_PER_W)])


_routing = pl.kernel(
    _routing_body,
    out_type=[
        jax.ShapeDtypeStruct((N_TOKENS * NUM_EXPERTS,), jnp.float32),
        jax.ShapeDtypeStruct((TOP_K * N_TOKENS,), jnp.float32),
        jax.ShapeDtypeStruct((TOP_K * N_TOKENS,), jnp.int32),
    ],
    mesh=plsc.VectorSubcoreMesh(core_axis_name="c", subcore_axis_name="s",
                                num_cores=_NC, num_subcores=_NS),
    scratch_types=[
        pltpu.VMEM((_ROWS_PER_W * NUM_EXPERTS,), jnp.float32),
        pltpu.VMEM((_ROWS_PER_W * NUM_EXPERTS,), jnp.float32),
        pltpu.VMEM((TOP_K * _ROWS_PER_W,), jnp.float32),
        pltpu.VMEM((TOP_K * _ROWS_PER_W,), jnp.int32),
    ],
    compiler_params=pltpu.CompilerParams(needs_layout_passes=False),
)


@jax.jit
def kernel(x, W):
    B, S, H = x.shape
    N = B * S
    x2 = x.reshape(N, H)
    wt = W.T

    logits = pl.pallas_call(
        _matmul_body,
        in_specs=[
            pl.BlockSpec(memory_space=pl.ANY),
            pl.BlockSpec((H, NUM_EXPERTS), lambda: (0, 0)),
        ],
        out_specs=pl.BlockSpec(memory_space=pl.ANY),
        out_shape=jax.ShapeDtypeStruct((N, NUM_EXPERTS), jnp.float32),
        scratch_shapes=[
            pltpu.VMEM((NBUF, CHUNK, HIDDEN), jnp.float32),
            pltpu.SemaphoreType.DMA((NBUF, NSPLIT)),
            pltpu.VMEM((NOUT, CHUNK, NUM_EXPERTS), jnp.float32),
            pltpu.SemaphoreType.DMA((NOUT,)),
        ],
    )(x2, wt)

    probs_flat, w_flat, i_flat = _routing(logits.reshape(-1))
    probs = probs_flat.reshape(N, NUM_EXPERTS)
    routing_weights = w_flat.reshape(B, S, TOP_K)
    expert_indices = i_flat.reshape(B, S, TOP_K)
    return (routing_weights, expert_indices, logits, probs)
